# baseline (speedup 1.0000x reference)
import functools

import jax
import jax.numpy as jnp
from jax import lax
from jax.experimental import pallas as pl
from jax.experimental.pallas import tpu as pltpu
from jax.experimental.pallas import tpu_sc as plsc

D = 64
S = 20
NUM_TOKENS = 1_000_000
NC = 2
NS = 16
NW = NC * NS
BSUB = 64
LANES = 16
NBG = BSUB // LANES


def _w2v_body(idx_in_hbm, idx_out_hbm, w_in_flat_hbm, w_out_hbm, out_hbm,
              in_idx, emb_idx, emb_buf, ctx_idx, ctx_buf, out_buf, sem):
    wid = lax.axis_index("s") * NC + lax.axis_index("c")
    lane = lax.iota(jnp.int32, LANES)
    bw = 512
    nsub = bw // BSUB
    nrow = BSUB * S

    def subtile(st, carry):
        b0 = wid * bw + st * BSUB
        pltpu.sync_copy(idx_in_hbm.at[pl.ds(b0, BSUB)], in_idx)
        for j in range(nrow // 128):
            pltpu.sync_copy(idx_out_hbm.at[pl.ds(b0 * S + j * 128, 128)],
                            ctx_idx.at[j])
        for g in range(NBG):
            iv = in_idx[pl.ds(g * LANES, LANES)]
            for d in range(D):
                k = d * BSUB + g * LANES
                emb_idx[k // 128, pl.ds(k % 128, LANES)] = iv + d * NUM_TOKENS
        copies = []
        for j in range(BSUB * D // 128):
            copies.append(pltpu.async_copy(
                w_in_flat_hbm.at[emb_idx.at[j]],
                emb_buf.at[pl.ds(j * 128, 128)], sem))
        for j in range(nrow // 128):
            copies.append(pltpu.async_copy(
                w_out_hbm.at[ctx_idx.at[j]],
                ctx_buf.at[pl.ds(j * 128, 128)], sem))
        for c in copies:
            c.wait()
        for bg in range(NBG):
            rows = [lane * S + (bg * LANES * S + s) for s in range(S)]

            def dbody(d, acc, rows=rows, bg=bg):
                emb_v = emb_buf[pl.ds(d * BSUB + bg * LANES, LANES)]
                dv = jnp.full((LANES,), d, jnp.int32)
                return tuple(
                    acc[s] + emb_v * plsc.load_gather(ctx_buf, [rows[s], dv])
                    for s in range(S))

            acc = lax.fori_loop(
                0, D, dbody,
                tuple(jnp.zeros((LANES,), jnp.float32) for _ in range(S)))
            for s in range(S):
                plsc.store_scatter(out_buf, [rows[s]], acc[s])
        pltpu.sync_copy(out_buf, out_hbm.at[pl.ds(b0 * S, nrow)])
        return carry

    lax.fori_loop(0, nsub, subtile, 0)


@jax.jit
def kernel(input_index_batch, output_indices_batch, W_in, W_out):
    B, s = output_indices_batch.shape
    mesh = plsc.VectorSubcoreMesh(
        core_axis_name="c", subcore_axis_name="s",
        num_cores=NC, num_subcores=NS)
    run = pl.kernel(
        _w2v_body, mesh=mesh,
        out_type=jax.ShapeDtypeStruct((B * S,), jnp.float32),
        scratch_types=[
            pltpu.VMEM((BSUB,), jnp.int32),
            pltpu.VMEM((BSUB * D // 128, 128), jnp.int32),
            pltpu.VMEM((BSUB * D,), jnp.float32),
            pltpu.VMEM((BSUB * S // 128, 128), jnp.int32),
            pltpu.VMEM((BSUB * S, D), jnp.float32),
            pltpu.VMEM((BSUB * S,), jnp.float32),
            pltpu.SemaphoreType.DMA,
        ],
        compiler_params=pltpu.CompilerParams(
            needs_layout_passes=False, use_tc_tiling_on_sc=False),
    )
    out = run(input_index_batch.astype(jnp.int32),
              output_indices_batch.reshape(-1).astype(jnp.int32),
              W_in.reshape(-1),
              W_out)
    return out.reshape(B, S)
